# Initial kernel scaffold; baseline (speedup 1.0000x reference)
import jax
import jax.numpy as jnp
from jax.experimental import pallas as pl


def kernel(input_class_prob, input_mask, target_mask, target_class, target_sizes):
    raise NotImplementedError("write your pallas kernel here")


# speedup vs baseline: 2.4319x; 2.4319x over previous
import functools

import jax
import jax.numpy as jnp
from jax.experimental import pallas as pl
from jax.experimental.pallas import tpu as pltpu

_EPS = 1e-5


def _matcher_body(cls_ref, tcls_ref, imask_ref, tmask_ref, out_ref, acc_ref,
                  *, n_chunks):
    j = pl.program_id(1)

    @pl.when(j == 0)
    def _init():
        acc_ref[...] = jnp.zeros_like(acc_ref)

    im = imask_ref[0]
    tm = tmask_ref[0]
    ones = jnp.ones((8, im.shape[1]), jnp.float32)
    lhs = jnp.concatenate([im, ones], axis=0)
    rhs = jnp.concatenate([tm, ones], axis=0)
    acc_ref[...] += jax.lax.dot_general(
        lhs, rhs, (((1,), (1,)), ((), ())),
        preferred_element_type=jnp.float32)

    @pl.when(j == n_chunks - 1)
    def _finish():
        n = out_ref.shape[1]
        k = out_ref.shape[2]
        inter = acc_ref[:n, :k]
        isum = acc_ref[:n, k:k + 1]
        tsum = acc_ref[n:n + 1, :k]
        dice = (2.0 * inter + _EPS) / ((isum + tsum) + _EPS)
        cls = cls_ref[0]
        tc = tcls_ref[0]
        cid = jax.lax.broadcasted_iota(jnp.int32, (cls.shape[1], k), 0)
        onehot = jnp.where(cid == tc, 1.0, 0.0)
        sim_class = jax.lax.dot_general(
            cls, onehot, (((1,), (0,)), ((), ())),
            preferred_element_type=jnp.float32,
            precision=jax.lax.Precision.HIGHEST)
        out_ref[0] = sim_class * dice


def kernel(input_class_prob, input_mask, target_mask, target_class,
           target_sizes):
    del target_sizes
    B, N, C = input_class_prob.shape
    K = target_class.shape[-1]
    HW = input_mask.shape[-1]
    CHUNK = 8192
    if HW % CHUNK:
        CHUNK = HW
    n_chunks = HW // CHUNK

    C_pad = max(128, -(-C // 128) * 128)
    cls = jnp.zeros((B, N, C_pad), jnp.float32).at[:, :, :C].set(
        input_class_prob)
    tcls = target_class.astype(jnp.int32).reshape(B, 1, K)

    return pl.pallas_call(
        functools.partial(_matcher_body, n_chunks=n_chunks),
        grid=(B, n_chunks),
        in_specs=[
            pl.BlockSpec((1, N, C_pad), lambda b, j: (b, 0, 0)),
            pl.BlockSpec((1, 1, K), lambda b, j: (b, 0, 0)),
            pl.BlockSpec((1, N, CHUNK), lambda b, j: (b, 0, j)),
            pl.BlockSpec((1, K, CHUNK), lambda b, j: (b, 0, j)),
        ],
        out_specs=pl.BlockSpec((1, N, K), lambda b, j: (b, 0, 0)),
        out_shape=jax.ShapeDtypeStruct((B, N, K), jnp.float32),
        scratch_shapes=[pltpu.VMEM((N + 8, K + 8), jnp.float32)],
        compiler_params=pltpu.CompilerParams(
            dimension_semantics=("parallel", "arbitrary"),
        ),
    )(cls, tcls, input_mask, target_mask)
